# baseline (speedup 1.0000x reference)
import functools

import jax
import jax.numpy as jnp
from jax.experimental import pallas as pl
from jax.experimental.pallas import tpu as pltpu

VOCAB_N = 1_000_000
DIM_N = 128
TILE = 8_000
NT = VOCAB_N // TILE


def _fwd_kernel(x_ref, emb_ref, w_ref, b_ref, out_ref, c_ref, acc_ref):
    i = pl.program_id(0)
    row = x_ref[0] % 8
    e = emb_ref[pl.ds(row, 1), :]
    w = w_ref[0]
    t = jax.lax.dot_general(
        e, w, (((1,), (1,)), ((), ())),
        preferred_element_type=jnp.float32,
    )
    t = t + b_ref[0]
    out_ref[0] = t
    tmax = jnp.max(t)

    @pl.when(i == 0)
    def _init():
        acc_ref[0] = tmax
        acc_ref[1] = jnp.sum(jnp.exp(t - tmax))

    @pl.when(i > 0)
    def _update():
        m_old = acc_ref[0]
        s_old = acc_ref[1]
        m_new = jnp.maximum(m_old, tmax)
        acc_ref[0] = m_new
        acc_ref[1] = s_old * jnp.exp(m_old - m_new) + jnp.sum(jnp.exp(t - m_new))

    @pl.when(i == NT - 1)
    def _finish():
        c_ref[0, 0] = acc_ref[0] + jnp.log(acc_ref[1])


def _norm_kernel(l_ref, c_ref, o_ref):
    o_ref[...] = l_ref[...] - c_ref[0, 0]


@jax.jit
def _run(x, emb, W, b):
    x = x.astype(jnp.int32)
    w3 = W.reshape(NT, TILE, DIM_N)
    b3 = b.reshape(NT, 1, TILE)

    grid_spec = pltpu.PrefetchScalarGridSpec(
        num_scalar_prefetch=1,
        grid=(NT,),
        in_specs=[
            pl.BlockSpec((8, DIM_N), lambda i, xr: (xr[0] // 8, 0)),
            pl.BlockSpec((1, TILE, DIM_N), lambda i, xr: (i, 0, 0)),
            pl.BlockSpec((1, 1, TILE), lambda i, xr: (i, 0, 0)),
        ],
        out_specs=[
            pl.BlockSpec((1, 1, TILE), lambda i, xr: (i, 0, 0)),
            pl.BlockSpec(memory_space=pltpu.SMEM),
        ],
        scratch_shapes=[pltpu.SMEM((2,), jnp.float32)],
    )
    logits, c = pl.pallas_call(
        _fwd_kernel,
        grid_spec=grid_spec,
        out_shape=[
            jax.ShapeDtypeStruct((NT, 1, TILE), jnp.float32),
            jax.ShapeDtypeStruct((1, 1), jnp.float32),
        ],
    )(x, emb, w3, b3)

    out = pl.pallas_call(
        _norm_kernel,
        grid=(NT // 5,),
        in_specs=[
            pl.BlockSpec((5, 1, TILE), lambda i: (i, 0, 0)),
            pl.BlockSpec(memory_space=pltpu.SMEM),
        ],
        out_specs=pl.BlockSpec((5, 1, TILE), lambda i: (i, 0, 0)),
        out_shape=jax.ShapeDtypeStruct((NT, 1, TILE), jnp.float32),
    )(logits, c)
    return out.reshape(1, VOCAB_N)


def kernel(x, emb, W, b):
    return _run(x, emb, W, b)


# speedup vs baseline: 1.0791x; 1.0791x over previous
import functools

import jax
import jax.numpy as jnp
from jax.experimental import pallas as pl
from jax.experimental.pallas import tpu as pltpu

VOCAB_N = 1_000_000
DIM_N = 128
TILE = 20_000
NT = VOCAB_N // TILE


def _fwd_kernel(x_ref, emb_ref, w_ref, b_ref, out_ref, macc_ref):
    i = pl.program_id(0)
    row = x_ref[0] % 8
    e = emb_ref[pl.ds(row, 1), :]
    w = w_ref[0]
    t = jax.lax.dot_general(
        e, w, (((1,), (1,)), ((), ())),
        preferred_element_type=jnp.float32,
    )
    t = t + b_ref[0]
    out_ref[0] = t

    @pl.when(i == 0)
    def _init():
        macc_ref[0] = t

    @pl.when(i > 0)
    def _update():
        macc_ref[0] = jnp.maximum(macc_ref[0], t)


def _reduce_kernel(l_ref, macc_ref, c_ref, m_ref, s_ref):
    i = pl.program_id(0)

    @pl.when(i == 0)
    def _init():
        m_ref[0] = jnp.max(macc_ref[0])
        s_ref[0] = 0.0

    m = m_ref[0]
    s_ref[0] += jnp.sum(jnp.exp(l_ref[0] - m))

    @pl.when(i == NT - 1)
    def _finish():
        c_ref[0, 0] = m + jnp.log(s_ref[0])


def _norm_kernel(l_ref, c_ref, o_ref):
    o_ref[...] = l_ref[...] - c_ref[0, 0]


@jax.jit
def _run(x, emb, W, b):
    x = x.astype(jnp.int32)
    w3 = W.reshape(NT, TILE, DIM_N)
    b3 = b.reshape(NT, 1, TILE)

    grid_spec = pltpu.PrefetchScalarGridSpec(
        num_scalar_prefetch=1,
        grid=(NT,),
        in_specs=[
            pl.BlockSpec((8, DIM_N), lambda i, xr: (xr[0] // 8, 0)),
            pl.BlockSpec((1, TILE, DIM_N), lambda i, xr: (i, 0, 0)),
            pl.BlockSpec((1, 1, TILE), lambda i, xr: (i, 0, 0)),
        ],
        out_specs=[
            pl.BlockSpec((1, 1, TILE), lambda i, xr: (i, 0, 0)),
            pl.BlockSpec((1, 1, TILE), lambda i, xr: (0, 0, 0)),
        ],
    )
    logits, macc = pl.pallas_call(
        _fwd_kernel,
        grid_spec=grid_spec,
        out_shape=[
            jax.ShapeDtypeStruct((NT, 1, TILE), jnp.float32),
            jax.ShapeDtypeStruct((1, 1, TILE), jnp.float32),
        ],
    )(x, emb, w3, b3)

    c = pl.pallas_call(
        _reduce_kernel,
        grid=(NT,),
        in_specs=[
            pl.BlockSpec((1, 1, TILE), lambda i: (i, 0, 0)),
            pl.BlockSpec((1, 1, TILE), lambda i: (0, 0, 0)),
        ],
        out_specs=pl.BlockSpec(memory_space=pltpu.SMEM),
        out_shape=jax.ShapeDtypeStruct((1, 1), jnp.float32),
        scratch_shapes=[
            pltpu.SMEM((1,), jnp.float32),
            pltpu.SMEM((1,), jnp.float32),
        ],
    )(logits, macc)

    out = pl.pallas_call(
        _norm_kernel,
        grid=(NT // 2,),
        in_specs=[
            pl.BlockSpec((2, 1, TILE), lambda i: (i, 0, 0)),
            pl.BlockSpec(memory_space=pltpu.SMEM),
        ],
        out_specs=pl.BlockSpec((2, 1, TILE), lambda i: (i, 0, 0)),
        out_shape=jax.ShapeDtypeStruct((NT, 1, TILE), jnp.float32),
    )(logits, c)
    return out.reshape(1, VOCAB_N)


def kernel(x, emb, W, b):
    return _run(x, emb, W, b)
